# Initial kernel scaffold; baseline (speedup 1.0000x reference)
import jax
import jax.numpy as jnp
from jax.experimental import pallas as pl


def kernel(z, embedding):
    raise NotImplementedError("write your pallas kernel here")


# speedup vs baseline: 1.4617x; 1.4617x over previous
import jax
import jax.numpy as jnp
from jax.experimental import pallas as pl
from jax.experimental.pallas import tpu as pltpu
from jax.experimental.pallas import tpu_sc as plsc

_D = 64
_K = 4096
_BT = 256
_GW = 128
_COMMIT = 0.25


def _make_tc_body(nb, nt):
  def body(z_ref, z2_ref, embt_ref, e2_ref, idx_ref, loss_ref, ppl_ref,
           d_scr, counts, acc):
    i = pl.program_id(0)

    @pl.when(i == 0)
    def _init():
      counts[...] = jnp.zeros_like(counts)
      acc[...] = jnp.zeros_like(acc)

    z = z_ref[...]
    za = (2.0 * z).astype(jnp.bfloat16)
    m = jax.lax.dot_general(za, embt_ref[...],
                            dimension_numbers=(((1,), (0,)), ((), ())),
                            preferred_element_type=jnp.float32)
    d_scr[...] = (z2_ref[...] - m) + e2_ref[...]
    d = d_scr[...]
    dmin = jnp.min(d, axis=1, keepdims=True)
    iota = jax.lax.broadcasted_iota(jnp.int32, (_BT, _K), 1)
    idx = jnp.min(jnp.where(d == dmin, iota, _K), axis=1)
    idx_ref[0, 0, :] = idx
    counts[...] += jnp.sum(jnp.where(iota == idx[:, None], 1.0, 0.0),
                           axis=0, keepdims=True)
    acc[...] += jnp.sum(dmin, keepdims=True)

    @pl.when(i == nb - 1)
    def _finish():
      loss_ref[...] = _COMMIT * (acc[...] / float(nt * _D))
      p = counts[...] * (1.0 / nt)
      ppl_ref[...] = jnp.exp(-jnp.sum(p * jnp.log(p + 1e-10), keepdims=True))

  return body


def _sc_gather(emb_padded, idx2d, nt):
  width = emb_padded.shape[1]
  mesh = plsc.VectorSubcoreMesh(core_axis_name="core",
                                subcore_axis_name="subcore")

  @pl.kernel(out_type=jax.ShapeDtypeStruct((nt, width), emb_padded.dtype),
             mesh=mesh)
  def gather_kernel(emb_hbm, i_hbm, o_hbm):
    def body(i_vmem, o_vmem):
      pltpu.sync_copy(emb_hbm.at[i_vmem.at[0]], o_vmem)

    pltpu.emit_pipeline(
        body,
        grid=(nt // _GW,),
        in_specs=[pl.BlockSpec((1, _GW), lambda i: (0, i))],
        out_specs=[pl.BlockSpec((_GW, width), lambda i: (i, 0))],
        core_axis_name=("core", "subcore"),
        dimension_semantics=(pltpu.PARALLEL,),
    )(i_hbm, o_hbm)

  return gather_kernel(emb_padded, idx2d)


def kernel(z, embedding):
  orig_shape = z.shape
  z_flat = z.reshape(-1, _D)
  nt = z_flat.shape[0]
  nb = nt // _BT

  e2 = jnp.sum(embedding ** 2, axis=-1)[None, :]
  z2 = jnp.sum(z_flat ** 2, axis=-1, keepdims=True)
  emb_t = embedding.T

  idx3, loss, ppl = pl.pallas_call(
      _make_tc_body(nb, nt),
      grid=(nb,),
      in_specs=[
          pl.BlockSpec((_BT, _D), lambda i: (i, 0)),
          pl.BlockSpec((_BT, 1), lambda i: (i, 0)),
          pl.BlockSpec((_D, _K), lambda i: (0, 0)),
          pl.BlockSpec((1, _K), lambda i: (0, 0)),
      ],
      out_specs=[
          pl.BlockSpec((1, 1, _BT), lambda i: (i, 0, 0)),
          pl.BlockSpec((1, 1), lambda i: (0, 0)),
          pl.BlockSpec((1, 1), lambda i: (0, 0)),
      ],
      out_shape=[
          jax.ShapeDtypeStruct((nb, 1, _BT), jnp.int32),
          jax.ShapeDtypeStruct((1, 1), jnp.float32),
          jax.ShapeDtypeStruct((1, 1), jnp.float32),
      ],
      scratch_shapes=[pltpu.VMEM((_BT, _K), jnp.float32),
                      pltpu.VMEM((1, _K), jnp.float32),
                      pltpu.VMEM((1, 1), jnp.float32)],
  )(z_flat, z2, emb_t, e2)

  indices = idx3.reshape(nt)
  emb_padded = jnp.pad(embedding, ((0, 0), (0, 128 - _D)))
  z_q = _sc_gather(emb_padded, indices.reshape(1, nt), nt)[:, :_D]
  z_q_st = z_flat + (z_q - z_flat)
  return (z_q_st.reshape(orig_shape), indices.reshape(orig_shape[:-1]),
          loss[0, 0], ppl[0, 0])
